# baseline (device time: 30514 ns/iter reference)
import jax
import jax.numpy as jnp
from jax import lax
from jax.experimental import pallas as pl
from jax.experimental.pallas import tpu as pltpu

N_DEV = 8
B, SQ, SKV = 2, 256, 256
HQ_PER, DH = 4, 64
CHUNK = HQ_PER * DH
BSQ = B * SQ
D_MODEL = 512
WINDOW = 128

F32 = jnp.float32
BF16 = jnp.bfloat16


def kernel(x, Wq, K_ext, V_ext, Wo):
    Wq_blk = Wq.reshape(D_MODEL, N_DEV, CHUNK).transpose(1, 0, 2)
    Wo_blk = Wo.reshape(N_DEV, CHUNK, D_MODEL)

    def body(x_ref, wq_ref, k_ref, v_ref, wo_ref, out_ref,
             ctx_ref, recv_ref, send_sems, recv_sems):
        my = lax.axis_index("i")

        barrier = pltpu.get_barrier_semaphore()
        for p in range(N_DEV):
            @pl.when(p != my)
            def _():
                pl.semaphore_signal(
                    barrier, inc=1,
                    device_id=(p,), device_id_type=pl.DeviceIdType.MESH,
                )
        pl.semaphore_wait(barrier, N_DEV - 1)

        x2d = x_ref[...].reshape(BSQ, D_MODEL).astype(BF16)
        wq_my = wq_ref[my].astype(BF16)
        q2d = jnp.dot(x2d, wq_my, preferred_element_type=F32)

        qi = lax.broadcasted_iota(jnp.int32, (SQ, SKV), 0)
        ki = lax.broadcasted_iota(jnp.int32, (SQ, SKV), 1)
        band = jnp.abs(qi - ki) <= WINDOW

        for b in range(B):
            qb = q2d[b * SQ:(b + 1) * SQ, :]
            for h in range(HQ_PER):
                q = qb[:, h * DH:(h + 1) * DH].astype(BF16)
                k = k_ref[b, :, h, :].astype(BF16)
                v = v_ref[b, :, h, :].astype(BF16)
                s = lax.dot_general(
                    q, k, (((1,), (1,)), ((), ())),
                    preferred_element_type=F32,
                ) * 0.125
                s = jnp.where(band, s, -1e9)
                m = jnp.max(s, axis=-1, keepdims=True)
                w = jnp.exp(s - m)
                w = (w / jnp.sum(w, axis=-1, keepdims=True)).astype(BF16)
                ctx = jnp.dot(w, v, preferred_element_type=F32)
                ctx_ref[b * SQ:(b + 1) * SQ, h * DH:(h + 1) * DH] = (
                    ctx.astype(BF16))

        for p in range(N_DEV):
            @pl.when(p != my)
            def _():
                rdma = pltpu.make_async_remote_copy(
                    src_ref=ctx_ref,
                    dst_ref=recv_ref.at[my],
                    send_sem=send_sems.at[p],
                    recv_sem=recv_sems.at[my],
                    device_id=(p,),
                    device_id_type=pl.DeviceIdType.MESH,
                )
                rdma.start()

        acc = jnp.dot(ctx_ref[...], wo_ref[my].astype(BF16),
                      preferred_element_type=F32)

        for p in range(N_DEV):
            @pl.when(p != my)
            def _():
                recv = pltpu.make_async_remote_copy(
                    src_ref=ctx_ref,
                    dst_ref=recv_ref.at[p],
                    send_sem=send_sems.at[p],
                    recv_sem=recv_sems.at[p],
                    device_id=(p,),
                    device_id_type=pl.DeviceIdType.MESH,
                )
                recv.wait_recv()
            contrib = jnp.dot(recv_ref[p], wo_ref[p].astype(BF16),
                              preferred_element_type=F32)
            acc = acc + jnp.where(p == my, 0.0, contrib)

        out_ref[...] = acc.reshape(B, SQ, D_MODEL)

        for p in range(N_DEV):
            @pl.when(p != my)
            def _():
                sent = pltpu.make_async_remote_copy(
                    src_ref=ctx_ref,
                    dst_ref=recv_ref.at[p],
                    send_sem=send_sems.at[p],
                    recv_sem=recv_sems.at[p],
                    device_id=(p,),
                    device_id_type=pl.DeviceIdType.MESH,
                )
                sent.wait_send()

    return pl.pallas_call(
        body,
        out_shape=jax.ShapeDtypeStruct((B, SQ, D_MODEL), F32),
        in_specs=[pl.BlockSpec(memory_space=pltpu.VMEM)] * 5,
        out_specs=pl.BlockSpec(memory_space=pltpu.VMEM),
        scratch_shapes=[
            pltpu.VMEM((BSQ, CHUNK), BF16),
            pltpu.VMEM((N_DEV, BSQ, CHUNK), BF16),
            pltpu.SemaphoreType.DMA((N_DEV,)),
            pltpu.SemaphoreType.DMA((N_DEV,)),
        ],
        compiler_params=pltpu.CompilerParams(collective_id=0),
    )(x, Wq_blk, K_ext, V_ext, Wo_blk)


# device time: 26845 ns/iter; 1.1367x vs baseline; 1.1367x over previous
import jax
import jax.numpy as jnp
from jax import lax
from jax.experimental import pallas as pl
from jax.experimental.pallas import tpu as pltpu

N_DEV = 8
B, SQ, SKV = 2, 256, 256
HQ_PER, DH = 4, 64
CHUNK = HQ_PER * DH
BSQ = B * SQ
SEG = BSQ // N_DEV
D_MODEL = 512
WINDOW = 128

F32 = jnp.float32
BF16 = jnp.bfloat16


def kernel(x, Wq, K_ext, V_ext, Wo):
    Wq_blk = Wq.reshape(D_MODEL, N_DEV, CHUNK).transpose(1, 0, 2)
    Wo_blk = Wo.reshape(N_DEV, CHUNK, D_MODEL)

    def body(x_ref, wq_ref, k_ref, v_ref, wo_ref, out_ref,
             ctx_ref, part_ref, red_ref, rs_recv, ag_recv,
             rs_send_sems, rs_recv_sems, ag_send_sems, ag_recv_sems):
        my = lax.axis_index("i")

        barrier = pltpu.get_barrier_semaphore()
        for p in range(N_DEV):
            @pl.when(p != my)
            def _():
                pl.semaphore_signal(
                    barrier, inc=1,
                    device_id=(p,), device_id_type=pl.DeviceIdType.MESH,
                )
        pl.semaphore_wait(barrier, N_DEV - 1)

        x2d = x_ref[...].reshape(BSQ, D_MODEL).astype(BF16)
        wq_my = wq_ref[my].astype(BF16)
        q2d = jnp.dot(x2d, wq_my, preferred_element_type=F32)

        qi = lax.broadcasted_iota(jnp.int32, (SQ, SKV), 0)
        ki = lax.broadcasted_iota(jnp.int32, (SQ, SKV), 1)
        band = jnp.abs(qi - ki) <= WINDOW

        for b in range(B):
            qb = q2d[b * SQ:(b + 1) * SQ, :]
            for h in range(HQ_PER):
                q = qb[:, h * DH:(h + 1) * DH].astype(BF16)
                k = k_ref[b, :, h, :].astype(BF16)
                v = v_ref[b, :, h, :].astype(BF16)
                s = lax.dot_general(
                    q, k, (((1,), (1,)), ((), ())),
                    preferred_element_type=F32,
                ) * 0.125
                s = jnp.where(band, s, -1e9)
                m = jnp.max(s, axis=-1, keepdims=True)
                w = jnp.exp(s - m)
                w = (w / jnp.sum(w, axis=-1, keepdims=True)).astype(BF16)
                ctx = jnp.dot(w, v, preferred_element_type=F32)
                ctx_ref[b * SQ:(b + 1) * SQ, h * DH:(h + 1) * DH] = (
                    ctx.astype(BF16))

        part = jnp.dot(ctx_ref[...], wo_ref[my].astype(BF16),
                       preferred_element_type=F32)
        part_ref[...] = part.astype(BF16)

        for p in range(N_DEV):
            @pl.when(p != my)
            def _():
                rdma = pltpu.make_async_remote_copy(
                    src_ref=part_ref.at[pl.ds(p * SEG, SEG)],
                    dst_ref=rs_recv.at[my],
                    send_sem=rs_send_sems.at[p],
                    recv_sem=rs_recv_sems.at[my],
                    device_id=(p,),
                    device_id_type=pl.DeviceIdType.MESH,
                )
                rdma.start()

        acc = part_ref[pl.ds(my * SEG, SEG), :].astype(F32)
        for p in range(N_DEV):
            @pl.when(p != my)
            def _():
                recv = pltpu.make_async_remote_copy(
                    src_ref=red_ref,
                    dst_ref=rs_recv.at[p],
                    send_sem=rs_send_sems.at[p],
                    recv_sem=rs_recv_sems.at[p],
                    device_id=(p,),
                    device_id_type=pl.DeviceIdType.MESH,
                )
                recv.wait_recv()
            acc = acc + jnp.where(p == my, 0.0, rs_recv[p].astype(F32))
        red_ref[...] = acc.astype(BF16)

        for p in range(N_DEV):
            @pl.when(p != my)
            def _():
                rdma = pltpu.make_async_remote_copy(
                    src_ref=red_ref,
                    dst_ref=ag_recv.at[my],
                    send_sem=ag_send_sems.at[p],
                    recv_sem=ag_recv_sems.at[my],
                    device_id=(p,),
                    device_id_type=pl.DeviceIdType.MESH,
                )
                rdma.start()

        out_ref[pl.ds(my // 4, 1), pl.ds((my % 4) * SEG, SEG), :] = (
            acc.reshape(1, SEG, D_MODEL))

        for s in range(N_DEV):
            @pl.when(s != my)
            def _():
                recv = pltpu.make_async_remote_copy(
                    src_ref=red_ref,
                    dst_ref=ag_recv.at[s],
                    send_sem=ag_send_sems.at[s],
                    recv_sem=ag_recv_sems.at[s],
                    device_id=(s,),
                    device_id_type=pl.DeviceIdType.MESH,
                )
                recv.wait_recv()
                out_ref[s // 4, (s % 4) * SEG:(s % 4 + 1) * SEG, :] = (
                    ag_recv[s].astype(F32))

        for sems in (rs_send_sems, ag_send_sems):
            for p in range(N_DEV):
                @pl.when(p != my)
                def _():
                    sent = pltpu.make_async_remote_copy(
                        src_ref=red_ref,
                        dst_ref=ag_recv.at[p],
                        send_sem=sems.at[p],
                        recv_sem=ag_recv_sems.at[p],
                        device_id=(p,),
                        device_id_type=pl.DeviceIdType.MESH,
                    )
                    sent.wait_send()

    return pl.pallas_call(
        body,
        out_shape=jax.ShapeDtypeStruct((B, SQ, D_MODEL), F32),
        in_specs=[pl.BlockSpec(memory_space=pltpu.VMEM)] * 5,
        out_specs=pl.BlockSpec(memory_space=pltpu.VMEM),
        scratch_shapes=[
            pltpu.VMEM((BSQ, CHUNK), BF16),
            pltpu.VMEM((BSQ, D_MODEL), BF16),
            pltpu.VMEM((SEG, D_MODEL), BF16),
            pltpu.VMEM((N_DEV, SEG, D_MODEL), BF16),
            pltpu.VMEM((N_DEV, SEG, D_MODEL), BF16),
            pltpu.SemaphoreType.DMA((N_DEV,)),
            pltpu.SemaphoreType.DMA((N_DEV,)),
            pltpu.SemaphoreType.DMA((N_DEV,)),
            pltpu.SemaphoreType.DMA((N_DEV,)),
        ],
        compiler_params=pltpu.CompilerParams(collective_id=0),
    )(x, Wq_blk, K_ext, V_ext, Wo_blk)
